# baseline (device time: 88155 ns/iter reference)
import jax
import jax.numpy as jnp
from jax import lax
from jax.experimental import pallas as pl
from jax.experimental.pallas import tpu as pltpu

N_DEV = 16
LOG2_N = 4


def kernel(x, Wq, Wo, K_ext, V_ext):
    B, Sq, D = x.shape
    _, Skv, Hkv, Dh = K_ext.shape
    Dq = Wq.shape[1]
    Hq = Dq // Dh
    G = Hq // Hkv
    Do = Wo.shape[1]
    NL = B * Hq

    def body(x_ref, wq_ref, wo_ref, k_ref, v_ref, out_ref,
             o_acc, l_acc, o_rx, l_rx, o_ssem, o_rsem, l_ssem, l_rsem):
        me = lax.axis_index("i")

        bar = pltpu.get_barrier_semaphore()
        for r in range(LOG2_N):
            p = jnp.bitwise_xor(me, 1 << r)
            pl.semaphore_signal(bar, inc=1, device_id=(p,),
                                device_id_type=pl.DeviceIdType.MESH)
        pl.semaphore_wait(bar, LOG2_N)

        for b in range(B):
            xb = x_ref[b].astype(jnp.bfloat16)
            q = jnp.dot(xb, wq_ref[...].astype(jnp.bfloat16),
                        preferred_element_type=jnp.float32)
            kb = k_ref[b].reshape(Skv, Hkv * Dh).astype(jnp.bfloat16)
            vb = v_ref[b].reshape(Skv, Hkv * Dh).astype(jnp.bfloat16)
            for h in range(Hq):
                g = h // G
                qh = q[:, h * Dh:(h + 1) * Dh].astype(jnp.bfloat16)
                kh = kb[:, g * Dh:(g + 1) * Dh]
                vh = vb[:, g * Dh:(g + 1) * Dh]
                s = lax.dot_general(qh, kh, (((1,), (1,)), ((), ())),
                                    preferred_element_type=jnp.float32)
                p_ = jnp.exp(s * 0.125)
                l_acc[:, b * Hq + h:b * Hq + h + 1] = jnp.sum(
                    p_, axis=1, keepdims=True)
                o_acc[b, :, h * Dh:(h + 1) * Dh] = jnp.dot(
                    p_.astype(jnp.bfloat16), vh,
                    preferred_element_type=jnp.float32)

        for r in range(LOG2_N):
            p = jnp.bitwise_xor(me, 1 << r)
            o_rdma = pltpu.make_async_remote_copy(
                src_ref=o_acc, dst_ref=o_rx.at[r],
                send_sem=o_ssem.at[r], recv_sem=o_rsem.at[r],
                device_id=(p,), device_id_type=pl.DeviceIdType.MESH)
            l_rdma = pltpu.make_async_remote_copy(
                src_ref=l_acc, dst_ref=l_rx.at[r],
                send_sem=l_ssem.at[r], recv_sem=l_rsem.at[r],
                device_id=(p,), device_id_type=pl.DeviceIdType.MESH)
            o_rdma.start()
            l_rdma.start()
            o_rdma.wait()
            l_rdma.wait()
            o_acc[...] = o_acc[...] + o_rx[r]
            l_acc[...] = l_acc[...] + l_rx[r]

        wo = wo_ref[...].astype(jnp.bfloat16)
        for b in range(B):
            o = o_acc[b]
            blocks = []
            for h in range(Hq):
                lcol = l_acc[:, b * Hq + h:b * Hq + h + 1]
                blocks.append(o[:, h * Dh:(h + 1) * Dh] / lcol)
            onorm = jnp.concatenate(blocks, axis=1).astype(jnp.bfloat16)
            out_ref[b] = jnp.dot(onorm, wo,
                                 preferred_element_type=jnp.float32)

    return pl.pallas_call(
        body,
        out_shape=jax.ShapeDtypeStruct((B, Sq, Do), jnp.float32),
        in_specs=[pl.BlockSpec(memory_space=pltpu.VMEM)] * 5,
        out_specs=pl.BlockSpec(memory_space=pltpu.VMEM),
        scratch_shapes=[
            pltpu.VMEM((B, Sq, Dq), jnp.float32),
            pltpu.VMEM((Sq, NL), jnp.float32),
            pltpu.VMEM((LOG2_N, B, Sq, Dq), jnp.float32),
            pltpu.VMEM((LOG2_N, Sq, NL), jnp.float32),
            pltpu.SemaphoreType.DMA((LOG2_N,)),
            pltpu.SemaphoreType.DMA((LOG2_N,)),
            pltpu.SemaphoreType.DMA((LOG2_N,)),
            pltpu.SemaphoreType.DMA((LOG2_N,)),
        ],
        compiler_params=pltpu.CompilerParams(collective_id=0),
    )(x, Wq, Wo, K_ext, V_ext)


# device time: 44589 ns/iter; 1.9771x vs baseline; 1.9771x over previous
import jax
import jax.numpy as jnp
from jax import lax
from jax.experimental import pallas as pl
from jax.experimental.pallas import tpu as pltpu

N_DEV = 16
LOG2_N = 4
BLK = 32


def kernel(x, Wq, Wo, K_ext, V_ext):
    B, Sq, D = x.shape
    _, Skv, Hkv, Dh = K_ext.shape
    Dq = Wq.shape[1]
    Hq = Dq // Dh
    G = Hq // Hkv
    Do = Wo.shape[1]
    R = B * Sq
    RX_OFF = [0, 256, 384, 448]

    def body(x_ref, wq_ref, wo_ref, k_ref, v_ref, out_ref,
             o_acc, o_bf, o_tx, l_acc, o_rx, l_rx,
             o_ssem, o_rsem, l_ssem, l_rsem):
        me = lax.axis_index("i")
        pos = (((me & 1) << 3) | ((me & 2) << 1)
               | ((me & 4) >> 1) | ((me & 8) >> 3))

        bar = pltpu.get_barrier_semaphore()
        for r in range(LOG2_N):
            p = jnp.bitwise_xor(me, 1 << r)
            pl.semaphore_signal(bar, inc=1, device_id=(p,),
                                device_id_type=pl.DeviceIdType.MESH)
        pl.semaphore_wait(bar, LOG2_N)

        for b in range(B):
            xb = x_ref[b].astype(jnp.bfloat16)
            q = jnp.dot(xb, wq_ref[...].astype(jnp.bfloat16),
                        preferred_element_type=jnp.float32)
            kb = k_ref[b].reshape(Skv, Hkv * Dh).astype(jnp.bfloat16)
            vb = v_ref[b].reshape(Skv, Hkv * Dh).astype(jnp.bfloat16)
            for h in range(Hq):
                g = h // G
                qh = q[:, h * Dh:(h + 1) * Dh].astype(jnp.bfloat16)
                kh = kb[:, g * Dh:(g + 1) * Dh]
                vh = vb[:, g * Dh:(g + 1) * Dh]
                s = lax.dot_general(qh, kh, (((1,), (1,)), ((), ())),
                                    preferred_element_type=jnp.float32)
                p_ = jnp.exp(s * 0.125)
                l_acc[b * Sq:(b + 1) * Sq, h:h + 1] = jnp.sum(
                    p_, axis=1, keepdims=True)
                o_acc[b * Sq:(b + 1) * Sq, h * Dh:(h + 1) * Dh] = jnp.dot(
                    p_.astype(jnp.bfloat16), vh,
                    preferred_element_type=jnp.float32)

        for k in range(LOG2_N):
            p = jnp.bitwise_xor(me, 1 << k)
            nblk = 8 >> k
            rows = nblk * BLK
            s_keep = (pos >> (3 - k)) << (3 - k)
            s_send = jnp.bitwise_xor(s_keep, nblk)
            o_tx[0:rows, :] = o_acc[pl.ds(s_send * BLK, rows), :].astype(
                jnp.bfloat16)
            o_rdma = pltpu.make_async_remote_copy(
                src_ref=o_tx.at[pl.ds(0, rows)],
                dst_ref=o_rx.at[pl.ds(RX_OFF[k], rows)],
                send_sem=o_ssem.at[k], recv_sem=o_rsem.at[k],
                device_id=(p,), device_id_type=pl.DeviceIdType.MESH)
            l_rdma = pltpu.make_async_remote_copy(
                src_ref=l_acc.at[pl.ds(s_send * BLK, rows)],
                dst_ref=l_rx.at[pl.ds(RX_OFF[k], rows)],
                send_sem=l_ssem.at[k], recv_sem=l_rsem.at[k],
                device_id=(p,), device_id_type=pl.DeviceIdType.MESH)
            o_rdma.start()
            l_rdma.start()
            o_rdma.wait()
            l_rdma.wait()
            o_acc[pl.ds(s_keep * BLK, rows), :] = (
                o_acc[pl.ds(s_keep * BLK, rows), :]
                + o_rx[pl.ds(RX_OFF[k], rows), :].astype(jnp.float32))
            l_acc[pl.ds(s_keep * BLK, rows), :] = (
                l_acc[pl.ds(s_keep * BLK, rows), :]
                + l_rx[pl.ds(RX_OFF[k], rows), :])

        o_bf[pl.ds(pos * BLK, BLK), :] = o_acc[pl.ds(pos * BLK, BLK), :].astype(
            jnp.bfloat16)

        for k in range(LOG2_N):
            p = jnp.bitwise_xor(me, 1 << (3 - k))
            nblk = 1 << k
            rows = nblk * BLK
            s_mine = (pos >> k) << k
            o_rdma = pltpu.make_async_remote_copy(
                src_ref=o_bf.at[pl.ds(s_mine * BLK, rows)],
                dst_ref=o_bf.at[pl.ds(s_mine * BLK, rows)],
                send_sem=o_ssem.at[LOG2_N + k], recv_sem=o_rsem.at[LOG2_N + k],
                device_id=(p,), device_id_type=pl.DeviceIdType.MESH)
            l_rdma = pltpu.make_async_remote_copy(
                src_ref=l_acc.at[pl.ds(s_mine * BLK, rows)],
                dst_ref=l_acc.at[pl.ds(s_mine * BLK, rows)],
                send_sem=l_ssem.at[LOG2_N + k], recv_sem=l_rsem.at[LOG2_N + k],
                device_id=(p,), device_id_type=pl.DeviceIdType.MESH)
            o_rdma.start()
            l_rdma.start()
            o_rdma.wait()
            l_rdma.wait()

        wo = wo_ref[...].astype(jnp.bfloat16)
        for b in range(B):
            o = o_bf[b * Sq:(b + 1) * Sq, :].astype(jnp.float32)
            blocks = []
            for h in range(Hq):
                lcol = l_acc[b * Sq:(b + 1) * Sq, h:h + 1]
                blocks.append(o[:, h * Dh:(h + 1) * Dh] / lcol)
            onorm = jnp.concatenate(blocks, axis=1).astype(jnp.bfloat16)
            out_ref[b] = jnp.dot(onorm, wo,
                                 preferred_element_type=jnp.float32)

    return pl.pallas_call(
        body,
        out_shape=jax.ShapeDtypeStruct((B, Sq, Do), jnp.float32),
        in_specs=[pl.BlockSpec(memory_space=pltpu.VMEM)] * 5,
        out_specs=pl.BlockSpec(memory_space=pltpu.VMEM),
        scratch_shapes=[
            pltpu.VMEM((R, Dq), jnp.float32),
            pltpu.VMEM((R, Dq), jnp.bfloat16),
            pltpu.VMEM((Sq, Dq), jnp.bfloat16),
            pltpu.VMEM((R, Hq), jnp.float32),
            pltpu.VMEM((480, Dq), jnp.bfloat16),
            pltpu.VMEM((480, Hq), jnp.float32),
            pltpu.SemaphoreType.DMA((2 * LOG2_N,)),
            pltpu.SemaphoreType.DMA((2 * LOG2_N,)),
            pltpu.SemaphoreType.DMA((2 * LOG2_N,)),
            pltpu.SemaphoreType.DMA((2 * LOG2_N,)),
        ],
        compiler_params=pltpu.CompilerParams(collective_id=0),
    )(x, Wq, Wo, K_ext, V_ext)


# device time: 44312 ns/iter; 1.9894x vs baseline; 1.0063x over previous
import jax
import jax.numpy as jnp
from jax import lax
from jax.experimental import pallas as pl
from jax.experimental.pallas import tpu as pltpu

N_DEV = 16
LOG2_N = 4
BLK = 32


def kernel(x, Wq, Wo, K_ext, V_ext):
    B, Sq, D = x.shape
    _, Skv, Hkv, Dh = K_ext.shape
    Dq = Wq.shape[1]
    Hq = Dq // Dh
    G = Hq // Hkv
    Do = Wo.shape[1]
    R = B * Sq
    OFF = [0, 256, 384, 448]

    def body(x_ref, wq_ref, wo_ref, k_ref, v_ref, out_ref,
             o_acc, o_bf, l_acc, o_tx, l_tx, o_rx, l_rx,
             o_ssem, o_rsem, l_ssem, l_rsem):
        me = lax.axis_index("i")
        pos = (((me & 1) << 3) | ((me & 2) << 1)
               | ((me & 4) >> 1) | ((me & 8) >> 3))
        even = (me & 1) == 0
        pending = []

        bar = pltpu.get_barrier_semaphore()
        for r in range(LOG2_N):
            p = jnp.bitwise_xor(me, 1 << r)
            pl.semaphore_signal(bar, inc=1, device_id=(p,),
                                device_id_type=pl.DeviceIdType.MESH)
        pl.semaphore_wait(bar, LOG2_N)

        def compute_partial(b):
            xb = x_ref[b].astype(jnp.bfloat16)
            q = jnp.dot(xb, wq_ref[...].astype(jnp.bfloat16),
                        preferred_element_type=jnp.float32)
            kb = k_ref[b].reshape(Skv, Hkv * Dh).astype(jnp.bfloat16)
            vb = v_ref[b].reshape(Skv, Hkv * Dh).astype(jnp.bfloat16)
            for h in range(Hq):
                g = h // G
                qh = q[:, h * Dh:(h + 1) * Dh].astype(jnp.bfloat16)
                kh = kb[:, g * Dh:(g + 1) * Dh]
                vh = vb[:, g * Dh:(g + 1) * Dh]
                s = lax.dot_general(qh, kh, (((1,), (1,)), ((), ())),
                                    preferred_element_type=jnp.float32)
                p_ = jnp.exp(s * 0.125)
                l_acc[b * Sq:(b + 1) * Sq, h:h + 1] = jnp.sum(
                    p_, axis=1, keepdims=True)
                o_acc[b * Sq:(b + 1) * Sq, h * Dh:(h + 1) * Dh] = jnp.dot(
                    p_.astype(jnp.bfloat16), vh,
                    preferred_element_type=jnp.float32)

        def rs_round(k):
            p = jnp.bitwise_xor(me, 1 << k)
            nblk = 8 >> k
            rows = nblk * BLK
            s_keep = (pos >> (3 - k)) << (3 - k)
            s_send = jnp.bitwise_xor(s_keep, nblk)
            o_tx[OFF[k]:OFF[k] + rows, :] = o_acc[
                pl.ds(s_send * BLK, rows), :].astype(jnp.bfloat16)
            l_tx[OFF[k]:OFF[k] + rows, :] = l_acc[pl.ds(s_send * BLK, rows), :]
            o_rdma = pltpu.make_async_remote_copy(
                src_ref=o_tx.at[pl.ds(OFF[k], rows)],
                dst_ref=o_rx.at[pl.ds(OFF[k], rows)],
                send_sem=o_ssem.at[k], recv_sem=o_rsem.at[k],
                device_id=(p,), device_id_type=pl.DeviceIdType.MESH)
            l_rdma = pltpu.make_async_remote_copy(
                src_ref=l_tx.at[pl.ds(OFF[k], rows)],
                dst_ref=l_rx.at[pl.ds(OFF[k], rows)],
                send_sem=l_ssem.at[k], recv_sem=l_rsem.at[k],
                device_id=(p,), device_id_type=pl.DeviceIdType.MESH)
            o_rdma.start()
            l_rdma.start()
            pending.extend([o_rdma, l_rdma])
            return o_rdma, l_rdma, rows, s_keep

        def rs_finish(o_rdma, l_rdma, rows, s_keep):
            o_rdma.wait_recv()
            l_rdma.wait_recv()
            o_acc[pl.ds(s_keep * BLK, rows), :] = (
                o_acc[pl.ds(s_keep * BLK, rows), :]
                + o_rx[pl.ds(OFF_K(rows), rows), :].astype(jnp.float32))
            l_acc[pl.ds(s_keep * BLK, rows), :] = (
                l_acc[pl.ds(s_keep * BLK, rows), :]
                + l_rx[pl.ds(OFF_K(rows), rows), :])

        def OFF_K(rows):
            return {256: 0, 128: 256, 64: 384, 32: 448}[rows]

        @pl.when(even)
        def _():
            compute_partial(1)

        @pl.when(jnp.logical_not(even))
        def _():
            compute_partial(0)

        r0 = rs_round(0)

        @pl.when(even)
        def _():
            compute_partial(0)

        @pl.when(jnp.logical_not(even))
        def _():
            compute_partial(1)

        rs_finish(*r0)
        for k in range(1, LOG2_N):
            rs_finish(*rs_round(k))

        o_bf[pl.ds(pos * BLK, BLK), :] = o_acc[pl.ds(pos * BLK, BLK), :].astype(
            jnp.bfloat16)

        def ag_round(k):
            p = jnp.bitwise_xor(me, 1 << (3 - k))
            nblk = 1 << k
            rows = nblk * BLK
            s_mine = (pos >> k) << k
            o_rdma = pltpu.make_async_remote_copy(
                src_ref=o_bf.at[pl.ds(s_mine * BLK, rows)],
                dst_ref=o_bf.at[pl.ds(s_mine * BLK, rows)],
                send_sem=o_ssem.at[LOG2_N + k], recv_sem=o_rsem.at[LOG2_N + k],
                device_id=(p,), device_id_type=pl.DeviceIdType.MESH)
            l_rdma = pltpu.make_async_remote_copy(
                src_ref=l_acc.at[pl.ds(s_mine * BLK, rows)],
                dst_ref=l_acc.at[pl.ds(s_mine * BLK, rows)],
                send_sem=l_ssem.at[LOG2_N + k], recv_sem=l_rsem.at[LOG2_N + k],
                device_id=(p,), device_id_type=pl.DeviceIdType.MESH)
            o_rdma.start()
            l_rdma.start()
            pending.extend([o_rdma, l_rdma])
            return o_rdma, l_rdma

        for k in range(LOG2_N - 1):
            o_rdma, l_rdma = ag_round(k)
            o_rdma.wait_recv()
            l_rdma.wait_recv()

        def epilogue(b):
            wo = wo_ref[...].astype(jnp.bfloat16)
            o = o_bf[b * Sq:(b + 1) * Sq, :].astype(jnp.float32)
            blocks = []
            for h in range(Hq):
                lcol = l_acc[b * Sq:(b + 1) * Sq, h:h + 1]
                blocks.append(o[:, h * Dh:(h + 1) * Dh] / lcol)
            onorm = jnp.concatenate(blocks, axis=1).astype(jnp.bfloat16)
            out_ref[b] = jnp.dot(onorm, wo,
                                 preferred_element_type=jnp.float32)

        o_rdma, l_rdma = ag_round(LOG2_N - 1)

        @pl.when(even)
        def _():
            epilogue(0)

        @pl.when(jnp.logical_not(even))
        def _():
            epilogue(1)

        o_rdma.wait_recv()
        l_rdma.wait_recv()

        @pl.when(even)
        def _():
            epilogue(1)

        @pl.when(jnp.logical_not(even))
        def _():
            epilogue(0)

        for d in pending:
            d.wait_send()

    return pl.pallas_call(
        body,
        out_shape=jax.ShapeDtypeStruct((B, Sq, Do), jnp.float32),
        in_specs=[pl.BlockSpec(memory_space=pltpu.VMEM)] * 5,
        out_specs=pl.BlockSpec(memory_space=pltpu.VMEM),
        scratch_shapes=[
            pltpu.VMEM((R, Dq), jnp.float32),
            pltpu.VMEM((R, Dq), jnp.bfloat16),
            pltpu.VMEM((R, Hq), jnp.float32),
            pltpu.VMEM((480, Dq), jnp.bfloat16),
            pltpu.VMEM((480, Hq), jnp.float32),
            pltpu.VMEM((480, Dq), jnp.bfloat16),
            pltpu.VMEM((480, Hq), jnp.float32),
            pltpu.SemaphoreType.DMA((2 * LOG2_N,)),
            pltpu.SemaphoreType.DMA((2 * LOG2_N,)),
            pltpu.SemaphoreType.DMA((2 * LOG2_N,)),
            pltpu.SemaphoreType.DMA((2 * LOG2_N,)),
        ],
        compiler_params=pltpu.CompilerParams(collective_id=0),
    )(x, Wq, Wo, K_ext, V_ext)
